# Initial kernel scaffold; baseline (speedup 1.0000x reference)
import jax
import jax.numpy as jnp
from jax.experimental import pallas as pl


def kernel(x, edge_attr, edge_index, glob, batch, params):
    raise NotImplementedError("write your pallas kernel here")


# speedup vs baseline: 2.0190x; 2.0190x over previous
import functools

import jax
import jax.numpy as jnp
from jax import lax
from jax.experimental import pallas as pl
from jax.experimental.pallas import tpu as pltpu
from jax.experimental.pallas import tpu_sc as plsc

_NC, _NS = 2, 16
_NW = _NC * _NS
_C = 128
_F32 = jnp.float32


@functools.lru_cache(maxsize=None)
def _make_gather(N, H, R):
    trips = pl.cdiv(R, _NW)
    mesh = plsc.VectorSubcoreMesh(core_axis_name="c", subcore_axis_name="s")

    @functools.partial(
        pl.kernel,
        mesh=mesh,
        out_type=(
            jax.ShapeDtypeStruct((R * _C, H), _F32),
            jax.ShapeDtypeStruct((R * _C, H), _F32),
        ),
        scratch_types=[
            pltpu.VMEM((_C,), jnp.int32),
            pltpu.VMEM((_C,), jnp.int32),
            pltpu.VMEM((_C, H), _F32),
            pltpu.VMEM((_C, H), _F32),
            pltpu.SemaphoreType.DMA,
            pltpu.SemaphoreType.DMA,
        ],
    )
    def gather_k(table, rows2d, cols2d, outr, outc,
                 idxr, idxc, bufr, bufc, semr, semc):
        wid = lax.axis_index("s") * _NC + lax.axis_index("c")

        def body(t, carry):
            r = t * _NW + wid

            @pl.when(r < R)
            def _():
                pltpu.sync_copy(rows2d.at[r], idxr)
                pltpu.sync_copy(cols2d.at[r], idxc)
                cr = pltpu.async_copy(table.at[idxr], bufr, semr)
                cc = pltpu.async_copy(table.at[idxc], bufc, semc)
                cr.wait()
                cc.wait()
                pltpu.sync_copy(bufr, outr.at[pl.ds(r * _C, _C)])
                pltpu.sync_copy(bufc, outc.at[pl.ds(r * _C, _C)])

            return carry

        lax.fori_loop(0, trips, body, 0)

    return gather_k


@functools.lru_cache(maxsize=None)
def _make_scatter(N, H, R):
    trips = pl.cdiv(R, _NW)
    Np = ((N + 127) // 128) * 128
    srows = Np // _NS
    mesh = plsc.VectorSubcoreMesh(core_axis_name="c", subcore_axis_name="s")

    @functools.partial(
        pl.kernel,
        mesh=mesh,
        out_type=jax.ShapeDtypeStruct((_NC, Np, H), _F32),
        scratch_types=[
            pltpu.VMEM_SHARED((Np, H), _F32),
            pltpu.VMEM((_C,), jnp.int32),
            pltpu.VMEM((_C, H), _F32),
        ],
    )
    def scatter_k(vals, cols2d, zeros, out, acc, idx, buf):
        cid = lax.axis_index("c")
        sid = lax.axis_index("s")
        wid = sid * _NC + cid

        pltpu.sync_copy(zeros.at[pl.ds(sid * srows, srows)],
                        acc.at[pl.ds(sid * srows, srows)])
        plsc.subcore_barrier()

        def body(t, carry):
            r = t * _NW + wid

            @pl.when(r < R)
            def _():
                pltpu.sync_copy(cols2d.at[r], idx)
                pltpu.sync_copy(vals.at[pl.ds(r * _C, _C)], buf)
                pltpu.sync_copy(buf, acc.at[idx], add=True)

            return carry

        lax.fori_loop(0, trips, body, 0)
        plsc.subcore_barrier()
        pltpu.sync_copy(acc.at[pl.ds(sid * srows, srows)],
                        out.at[cid].at[pl.ds(sid * srows, srows)])

    return scatter_k


def _enc_body(x, w1, b1, a, w2, b2, o_ref):
    h = jnp.dot(x[...], w1[...], preferred_element_type=_F32) + b1[...]
    h = jnp.where(h >= 0, h, a[...] * h)
    o_ref[...] = jnp.dot(h, w2[...], preferred_element_type=_F32) + b2[...]


@functools.lru_cache(maxsize=None)
def _make_enc(M, din, dmid, dout, blk):
    return pl.pallas_call(
        _enc_body,
        grid=(M // blk,),
        in_specs=[
            pl.BlockSpec((blk, din), lambda i: (i, 0)),
            pl.BlockSpec((din, dmid), lambda i: (0, 0)),
            pl.BlockSpec((1, dmid), lambda i: (0, 0)),
            pl.BlockSpec((1, 1), lambda i: (0, 0)),
            pl.BlockSpec((dmid, dout), lambda i: (0, 0)),
            pl.BlockSpec((1, dout), lambda i: (0, 0)),
        ],
        out_specs=pl.BlockSpec((blk, dout), lambda i: (i, 0)),
        out_shape=jax.ShapeDtypeStruct((M, dout), _F32),
    )


def _ln_prelu(h, lnw, lnb, a):
    m = jnp.mean(h, axis=1, keepdims=True)
    v = jnp.mean((h - m) ** 2, axis=1, keepdims=True)
    h = (h - m) * lax.rsqrt(v + 1e-5) * lnw + lnb
    return jnp.where(h >= 0, h, a * h)


def _edge_body(xr, xc, ea, w1r, w1c, w1e, b1, lnw, lnb, a, w2, b2,
               eo_ref, ean_ref):
    h = (jnp.dot(xr[...], w1r[...], preferred_element_type=_F32)
         + jnp.dot(xc[...], w1c[...], preferred_element_type=_F32)
         + jnp.dot(ea[...], w1e[...], preferred_element_type=_F32)
         + b1[...])
    h = _ln_prelu(h, lnw[...], lnb[...], a[...])
    eo = jnp.dot(h, w2[...], preferred_element_type=_F32) + b2[...]
    eo_ref[...] = eo
    ean_ref[...] = ea[...] + eo


@functools.lru_cache(maxsize=None)
def _make_edge_mlp(E, H, blk):
    row = lambda i: (i, 0)
    const = lambda i: (0, 0)
    return pl.pallas_call(
        _edge_body,
        grid=(E // blk,),
        in_specs=[
            pl.BlockSpec((blk, H), row),
            pl.BlockSpec((blk, H), row),
            pl.BlockSpec((blk, H), row),
            pl.BlockSpec((H, 2 * H), const),
            pl.BlockSpec((H, 2 * H), const),
            pl.BlockSpec((H, 2 * H), const),
            pl.BlockSpec((1, 2 * H), const),
            pl.BlockSpec((1, 2 * H), const),
            pl.BlockSpec((1, 2 * H), const),
            pl.BlockSpec((1, 1), const),
            pl.BlockSpec((2 * H, H), const),
            pl.BlockSpec((1, H), const),
        ],
        out_specs=(
            pl.BlockSpec((blk, H), row),
            pl.BlockSpec((blk, H), row),
        ),
        out_shape=(
            jax.ShapeDtypeStruct((E, H), _F32),
            jax.ShapeDtypeStruct((E, H), _F32),
        ),
    )


def _node_body(x, a0, a1, wx, wa, b1, lnw, lnb, a, w2, b2, o_ref):
    agg = a0[...] + a1[...]
    h = (jnp.dot(x[...], wx[...], preferred_element_type=_F32)
         + jnp.dot(agg, wa[...], preferred_element_type=_F32)
         + b1[...])
    h = _ln_prelu(h, lnw[...], lnb[...], a[...])
    o_ref[...] = x[...] + jnp.dot(h, w2[...], preferred_element_type=_F32) + b2[...]


@functools.lru_cache(maxsize=None)
def _make_node_mlp(N, H, blk):
    row = lambda i: (i, 0)
    const = lambda i: (0, 0)
    return pl.pallas_call(
        _node_body,
        grid=(N // blk,),
        in_specs=[
            pl.BlockSpec((blk, H), row),
            pl.BlockSpec((blk, H), row),
            pl.BlockSpec((blk, H), row),
            pl.BlockSpec((H, 2 * H), const),
            pl.BlockSpec((H, 2 * H), const),
            pl.BlockSpec((1, 2 * H), const),
            pl.BlockSpec((1, 2 * H), const),
            pl.BlockSpec((1, 2 * H), const),
            pl.BlockSpec((1, 1), const),
            pl.BlockSpec((2 * H, H), const),
            pl.BlockSpec((1, H), const),
        ],
        out_specs=pl.BlockSpec((blk, H), row),
        out_shape=jax.ShapeDtypeStruct((N, H), _F32),
    )


def _dec_body(xp, bp, glob_ref, emb_ref,
              gw1, gb1, ga, gw2, gb2,
              wdu, wdn, db, alpha,
              ow1, ob1, olnw, olnb, oa, ow2, ob2,
              sph, spl, spb,
              dos_ref, spark_ref):
    B = dos_ref.shape[0]
    NB = xp.shape[0]
    NE, H = emb_ref.shape
    g = glob_ref[...]
    u = g[:, 0:1] * gw1[0:1, :] + g[:, 1:2] * gw1[1:2, :] + gb1[...]
    u = jnp.where(u >= 0, u, ga[...] * u)
    u = jnp.dot(u, gw2[...], preferred_element_type=_F32) + gb2[...]
    cls = lax.broadcasted_iota(jnp.int32, (B, 128), 0)

    def nb(i, acc):
        mf = (bp[i] == cls).astype(_F32)
        return acc + jnp.dot(mf, xp[i], preferred_element_type=_F32)

    node_sum = lax.fori_loop(0, NB, nb, jnp.zeros((B, H), _F32))
    graph = (jnp.dot(u, wdu[...], preferred_element_type=_F32)
             + jnp.dot(node_sum, wdn[...], preferred_element_type=_F32)
             + db[...])
    emb = emb_ref[...]
    z = emb[None, :, :] + (alpha[...] * graph)[:, None, :]
    h = jnp.dot(z.reshape(B * NE, H), ow1[...], preferred_element_type=_F32) + ob1[...]
    h = _ln_prelu(h, olnw[...], olnb[...], oa[...])
    dos_ref[...] = jnp.sum(h.reshape(B, NE, H) * ow2[...][None], axis=2) + ob2[...]
    v1 = jnp.sum(emb * sph[...], axis=1, keepdims=True)
    v2 = jnp.sum(emb * spl[...], axis=1, keepdims=True)
    logit = v1[1:, :] + v2[:-1, :] + spb[...]
    spark_ref[...] = jnp.broadcast_to(jax.nn.sigmoid(logit), spark_ref.shape)


@functools.lru_cache(maxsize=None)
def _make_decoder(NB, B, NE, H):
    return pl.pallas_call(
        _dec_body,
        out_shape=(
            jax.ShapeDtypeStruct((B, NE), _F32),
            jax.ShapeDtypeStruct((NE - 1, B), _F32),
        ),
    )


def _s11(v):
    return jnp.reshape(v, (1, 1)).astype(_F32)


def kernel(x, edge_attr, edge_index, glob, batch, params):
    p = params
    N, H = x.shape
    E, DE = edge_attr.shape
    B = glob.reshape(-1, 2).shape[0]
    R = E // _C

    row2d = edge_index[0].reshape(R, _C)
    col2d = edge_index[1].reshape(R, _C)

    ne, ee = p["node_enc"], p["edge_enc"]
    xc = _make_enc(N, H, H, H, 1000)(
        x, ne["l1"]["w"].T, ne["l1"]["b"][None], _s11(ne["a"]),
        ne["l2"]["w"].T, ne["l2"]["b"][None])
    ea = _make_enc(E, DE, H, H, 2000)(
        edge_attr, ee["l1"]["w"].T, ee["l1"]["b"][None], _s11(ee["a"]),
        ee["l2"]["w"].T, ee["l2"]["b"][None])

    gather = _make_gather(N, H, R)
    scatter = _make_scatter(N, H, R)
    edge_mlp = _make_edge_mlp(E, H, 512)
    node_mlp = _make_node_mlp(N, H, 1000)
    Np = ((N + 127) // 128) * 128
    zeros = jnp.zeros((Np, H), _F32)

    for lp in p["layers"]:
        xg_r, xg_c = gather(xc, row2d, col2d)
        em = lp["edge"]
        w1t = em["l1"]["w"].T
        eo, ea = edge_mlp(
            xg_r, xg_c, ea,
            w1t[:H], w1t[H:2 * H], w1t[2 * H:],
            em["l1"]["b"][None], em["ln_w"][None], em["ln_b"][None],
            _s11(em["a"]), em["l2"]["w"].T, em["l2"]["b"][None])
        part = scatter(eo, col2d, zeros)
        nm = lp["node"]
        wn = nm["l1"]["w"].T
        xc = node_mlp(
            xc, part[0, :N], part[1, :N],
            wn[:H], wn[H:],
            nm["l1"]["b"][None], nm["ln_w"][None], nm["ln_b"][None],
            _s11(nm["a"]), nm["l2"]["w"].T, nm["l2"]["b"][None])

    NB = pl.cdiv(N, 128)
    pad = NB * 128 - N
    xp = jnp.pad(xc, ((0, pad), (0, 0))).reshape(NB, 128, H)
    bp = jnp.pad(batch, (0, pad), constant_values=B).reshape(NB, 1, 128)
    ge, de, om = p["glob_enc"], p["dec"], p["out"]
    NE = p["emb"].shape[0]
    spw = p["spark"]["w"]
    dos, spark2 = _make_decoder(NB, B, NE, H)(
        xp, bp, glob.reshape(-1, 2), p["emb"],
        ge["l1"]["w"].T, ge["l1"]["b"][None], _s11(ge["a"]),
        ge["l2"]["w"].T, ge["l2"]["b"][None],
        de["w"].T[:H], de["w"].T[H:], de["b"][None], _s11(p["alpha"]),
        om["l1"]["w"].T, om["l1"]["b"][None], om["ln_w"][None],
        om["ln_b"][None], _s11(om["a"]),
        om["l2"]["w"], om["l2"]["b"][None],
        spw[:, :H], spw[:, H:], _s11(p["spark"]["b"]))
    return (dos, xc, spark2[:, :, None])
